# baseline (speedup 1.0000x reference)
import functools

import jax
import jax.numpy as jnp
from jax import lax
from jax.experimental import pallas as pl
from jax.experimental.pallas import tpu as pltpu
from jax.experimental.pallas import tpu_sc as plsc

N = 10000
E = 320000
D_HID = 128
D_OUT = 64
EPS = 1e-5

NC = 2
NS = 16
NW = NC * NS
B = 128
N_PAD = 10240
E_PAD = 327680
TPT = E_PAD // NW
NBLK = TPT // B
NBLK0 = 142
NBLK1 = 18
RPT = N_PAD // NS


def _zero_vmem(buf, rows, cols):
    z = jnp.zeros((16,), jnp.float32)
    for i in range(rows):
        for j in range(cols // 16):
            buf[i, pl.ds(j * 16, 16)] = z


def _make_deg():
    mesh = plsc.VectorSubcoreMesh(
        core_axis_name="c", subcore_axis_name="s", num_cores=NC,
        num_subcores=NS)

    @functools.partial(
        pl.kernel,
        out_type=jax.ShapeDtypeStruct((NC * N_PAD, 16), jnp.float32),
        mesh=mesh,
        compiler_params=pltpu.CompilerParams(use_tc_tiling_on_sc=False),
        scratch_types=[
            pltpu.VMEM((NBLK, B), jnp.int32),
            pltpu.VMEM((B, 16), jnp.float32),
            pltpu.VMEM((16, 16), jnp.float32),
            pltpu.VMEM_SHARED((N_PAD, 16), jnp.float32),
        ],
    )
    def deg_kernel(dst_hbm, out_hbm, didx, ones_v, zbuf, acc):
        cid = lax.axis_index("c")
        sid = lax.axis_index("s")
        wid = cid * NS + sid
        _zero_vmem(zbuf, 16, 16)
        one = jnp.full((16,), 1.0, jnp.float32)
        for i in range(B):
            ones_v[i, pl.ds(0, 16)] = one
        pltpu.sync_copy(dst_hbm.at[pl.ds(wid * NBLK, NBLK)], didx)

        def zrow(r, _):
            pltpu.sync_copy(zbuf, acc.at[pl.ds(sid * RPT + r * 16, 16)])
            return 0
        lax.fori_loop(0, RPT // 16, zrow, 0)
        plsc.subcore_barrier()

        def body(j, _):
            pltpu.sync_copy(ones_v, acc.at[didx.at[j]], add=True)
            return 0
        lax.fori_loop(0, NBLK, body, 0)
        plsc.subcore_barrier()
        pltpu.sync_copy(acc.at[pl.ds(sid * RPT, RPT)],
                        out_hbm.at[pl.ds(cid * N_PAD + sid * RPT, RPT)])

    return deg_kernel


def _make_prop(d):
    mesh = plsc.VectorSubcoreMesh(
        core_axis_name="c", subcore_axis_name="s", num_cores=NC,
        num_subcores=NS)

    @functools.partial(
        pl.kernel,
        out_type=jax.ShapeDtypeStruct((NC * N_PAD, d), jnp.float32),
        mesh=mesh,
        compiler_params=pltpu.CompilerParams(use_tc_tiling_on_sc=False),
        scratch_types=[
            pltpu.VMEM((2, B), jnp.int32),
            pltpu.VMEM((2, B), jnp.int32),
            pltpu.VMEM((B, d), jnp.float32),
            pltpu.VMEM((16, d), jnp.float32),
            pltpu.VMEM_SHARED((N_PAD, d), jnp.float32),
            pltpu.SemaphoreType.DMA,
            pltpu.SemaphoreType.DMA,
            pltpu.SemaphoreType.DMA,
        ],
    )
    def prop_kernel(idx_hbm, u_hbm, out_hbm,
                    ib0, ib1, rows, zbuf, acc, sem, isem0, isem1):
        cid = lax.axis_index("c")
        sid = lax.axis_index("s")
        _zero_vmem(zbuf, 16, d)

        def zrow(r, _):
            pltpu.sync_copy(zbuf, acc.at[pl.ds(sid * RPT + r * 16, 16)])
            return 0
        lax.fori_loop(0, RPT // 16, zrow, 0)
        plsc.subcore_barrier()

        tile_base = jnp.where(cid == 0, sid * NBLK0,
                              NS * NBLK0 + sid * NBLK1)
        nblk = jnp.where(cid == 0, NBLK0, NBLK1)

        pltpu.async_copy(idx_hbm.at[tile_base], ib0, isem0)

        def step(row, ib, isem, nib, nisem):
            pltpu.make_async_copy(idx_hbm.at[row], ib, isem).wait()
            pltpu.async_copy(idx_hbm.at[row + 1], nib, nisem)
            pltpu.async_copy(u_hbm.at[ib.at[0]], rows, sem).wait()
            pltpu.sync_copy(rows, acc.at[ib.at[1]], add=True)

        def body(k, _):
            row = tile_base + 2 * k
            step(row, ib0, isem0, ib1, isem1)
            step(row + 1, ib1, isem1, ib0, isem0)
            return 0
        lax.fori_loop(0, nblk // 2, body, 0)
        pltpu.make_async_copy(idx_hbm.at[tile_base], ib0, isem0).wait()
        plsc.subcore_barrier()
        pltpu.sync_copy(acc.at[pl.ds(sid * RPT, RPT)],
                        out_hbm.at[pl.ds(cid * N_PAD + sid * RPT, RPT)])

    return prop_kernel


_R = 1000


def _dinv(dp0, dp1):
    deg = 1.0 + dp0[:, 0:1] + dp1[:, 0:1]
    return lax.rsqrt(deg)


def _tc_first(x_ref, w_ref, dp0_ref, dp1_ref, u_ref):
    di = _dinv(dp0_ref[...], dp1_ref[...])
    u_ref[...] = di * jnp.dot(x_ref[...], w_ref[...],
                              preferred_element_type=jnp.float32)


def _tc_mid(s0_ref, s1_ref, u_ref, w_ref, b_ref, g_ref, be_ref,
            dp0_ref, dp1_ref, o_ref):
    di = _dinv(dp0_ref[...], dp1_ref[...])
    s = s0_ref[...] + s1_ref[...] + u_ref[...]
    out = di * s + b_ref[...]
    z = g_ref[...] * (out * (1.0 / jnp.sqrt(1.0 + EPS))) + be_ref[...]
    r = jnp.maximum(z, 0.0)
    o_ref[...] = di * jnp.dot(r, w_ref[...],
                              preferred_element_type=jnp.float32)


def _tc_last(s0_ref, s1_ref, u_ref, b_ref, dp0_ref, dp1_ref, o_ref):
    di = _dinv(dp0_ref[...], dp1_ref[...])
    o_ref[...] = di * (s0_ref[...] + s1_ref[...] + u_ref[...]) + b_ref[...]


def _row_spec(d):
    return pl.BlockSpec((_R, d), lambda i: (i, 0))


def _full_spec(r, c):
    return pl.BlockSpec((r, c), lambda i: (0, 0))


def kernel(x, edge_index, W1, b1, g1, be1, W2, b2, g2, be2, W3, b3):
    src = edge_index[0]
    dst = edge_index[1]
    pad = E_PAD - E
    src_p = jnp.concatenate(
        [src, jnp.zeros((pad,), jnp.int32)]).reshape(E_PAD // B, B)
    dst_pad = N + (jnp.arange(pad, dtype=jnp.int32) % (N_PAD - N))
    dst_p = jnp.concatenate([dst, dst_pad]).reshape(E_PAD // B, B)
    idx_p = jnp.concatenate(
        [jnp.stack([src_p, dst_p], axis=1),
         jnp.zeros((1, 2, B), jnp.int32)])

    deg_fn = _make_deg()
    prop128 = _make_prop(D_HID)
    prop64 = _make_prop(D_OUT)

    deg_out = deg_fn(dst_p)
    dp0 = deg_out[:N]
    dp1 = deg_out[N_PAD:N_PAD + N]

    grid = (N // _R,)
    dspec = pl.BlockSpec((_R, 16), lambda i: (i, 0))

    u1 = pl.pallas_call(
        _tc_first,
        grid=grid,
        in_specs=[_row_spec(D_HID), _full_spec(D_HID, D_HID), dspec, dspec],
        out_specs=_row_spec(D_HID),
        out_shape=jax.ShapeDtypeStruct((N, D_HID), jnp.float32),
    )(x, W1, dp0, dp1)

    p1 = prop128(idx_p, u1)
    u2 = pl.pallas_call(
        _tc_mid,
        grid=grid,
        in_specs=[_row_spec(D_HID), _row_spec(D_HID), _row_spec(D_HID),
                  _full_spec(D_HID, D_HID), _full_spec(1, D_HID),
                  _full_spec(1, D_HID), _full_spec(1, D_HID), dspec, dspec],
        out_specs=_row_spec(D_HID),
        out_shape=jax.ShapeDtypeStruct((N, D_HID), jnp.float32),
    )(p1[:N], p1[N_PAD:N_PAD + N], u1, W2, b1.reshape(1, -1),
      g1.reshape(1, -1), be1.reshape(1, -1), dp0, dp1)

    p2 = prop128(idx_p, u2)
    u3 = pl.pallas_call(
        _tc_mid,
        grid=grid,
        in_specs=[_row_spec(D_HID), _row_spec(D_HID), _row_spec(D_HID),
                  _full_spec(D_HID, D_OUT), _full_spec(1, D_HID),
                  _full_spec(1, D_HID), _full_spec(1, D_HID), dspec, dspec],
        out_specs=_row_spec(D_OUT),
        out_shape=jax.ShapeDtypeStruct((N, D_OUT), jnp.float32),
    )(p2[:N], p2[N_PAD:N_PAD + N], u2, W3, b2.reshape(1, -1),
      g2.reshape(1, -1), be2.reshape(1, -1), dp0, dp1)

    p3 = prop64(idx_p, u3)
    out = pl.pallas_call(
        _tc_last,
        grid=grid,
        in_specs=[_row_spec(D_OUT), _row_spec(D_OUT), _row_spec(D_OUT),
                  _full_spec(1, D_OUT), dspec, dspec],
        out_specs=_row_spec(D_OUT),
        out_shape=jax.ShapeDtypeStruct((N, D_OUT), jnp.float32),
    )(p3[:N], p3[N_PAD:N_PAD + N], u3, b3.reshape(1, -1), dp0, dp1)

    return out


# speedup vs baseline: 1.0094x; 1.0094x over previous
import functools

import jax
import jax.numpy as jnp
from jax import lax
from jax.experimental import pallas as pl
from jax.experimental.pallas import tpu as pltpu
from jax.experimental.pallas import tpu_sc as plsc

N = 10000
E = 320000
D_HID = 128
D_OUT = 64
EPS = 1e-5

NC = 2
NS = 16
NW = NC * NS
B = 128
N_PAD = 10240
E_PAD = 327680
TPT = E_PAD // NW
NBLK = TPT // B
NBLK0 = 134
NBLK1 = 26
RPT = N_PAD // NS


def _zero_vmem(buf, rows, cols):
    z = jnp.zeros((16,), jnp.float32)
    for i in range(rows):
        for j in range(cols // 16):
            buf[i, pl.ds(j * 16, 16)] = z


def _make_deg():
    mesh = plsc.VectorSubcoreMesh(
        core_axis_name="c", subcore_axis_name="s", num_cores=NC,
        num_subcores=NS)

    @functools.partial(
        pl.kernel,
        out_type=jax.ShapeDtypeStruct((NC * N_PAD, 16), jnp.float32),
        mesh=mesh,
        compiler_params=pltpu.CompilerParams(use_tc_tiling_on_sc=False),
        scratch_types=[
            pltpu.VMEM((NBLK, B), jnp.int32),
            pltpu.VMEM((B, 16), jnp.float32),
            pltpu.VMEM((16, 16), jnp.float32),
            pltpu.VMEM_SHARED((N_PAD, 16), jnp.float32),
        ],
    )
    def deg_kernel(dst_hbm, out_hbm, didx, ones_v, zbuf, acc):
        cid = lax.axis_index("c")
        sid = lax.axis_index("s")
        wid = cid * NS + sid
        _zero_vmem(zbuf, 16, 16)
        one = jnp.full((16,), 1.0, jnp.float32)
        for i in range(B):
            ones_v[i, pl.ds(0, 16)] = one
        pltpu.sync_copy(dst_hbm.at[pl.ds(wid * NBLK, NBLK)], didx)

        def zrow(r, _):
            pltpu.sync_copy(zbuf, acc.at[pl.ds(sid * RPT + r * 16, 16)])
            return 0
        lax.fori_loop(0, RPT // 16, zrow, 0)
        plsc.subcore_barrier()

        def body(j, _):
            pltpu.sync_copy(ones_v, acc.at[didx.at[j]], add=True)
            return 0
        lax.fori_loop(0, NBLK, body, 0)
        plsc.subcore_barrier()
        pltpu.sync_copy(acc.at[pl.ds(sid * RPT, RPT)],
                        out_hbm.at[pl.ds(cid * N_PAD + sid * RPT, RPT)])

    return deg_kernel


def _make_prop(d):
    mesh = plsc.VectorSubcoreMesh(
        core_axis_name="c", subcore_axis_name="s", num_cores=NC,
        num_subcores=NS)

    @functools.partial(
        pl.kernel,
        out_type=jax.ShapeDtypeStruct((NC * N_PAD, d), jnp.float32),
        mesh=mesh,
        compiler_params=pltpu.CompilerParams(use_tc_tiling_on_sc=False),
        scratch_types=[
            pltpu.VMEM((2, B), jnp.int32),
            pltpu.VMEM((2, B), jnp.int32),
            pltpu.VMEM((B, d), jnp.float32),
            pltpu.VMEM((16, d), jnp.float32),
            pltpu.VMEM_SHARED((N_PAD, d), jnp.float32),
            pltpu.SemaphoreType.DMA,
            pltpu.SemaphoreType.DMA,
            pltpu.SemaphoreType.DMA,
        ],
    )
    def prop_kernel(idx_hbm, u_hbm, out_hbm,
                    ib0, ib1, rows, zbuf, acc, sem, isem0, isem1):
        cid = lax.axis_index("c")
        sid = lax.axis_index("s")
        _zero_vmem(zbuf, 16, d)

        def zrow(r, _):
            pltpu.sync_copy(zbuf, acc.at[pl.ds(sid * RPT + r * 16, 16)])
            return 0
        lax.fori_loop(0, RPT // 16, zrow, 0)
        plsc.subcore_barrier()

        tile_base = jnp.where(cid == 0, sid * NBLK0,
                              NS * NBLK0 + sid * NBLK1)
        nblk = jnp.where(cid == 0, NBLK0, NBLK1)

        pltpu.async_copy(idx_hbm.at[tile_base], ib0, isem0)

        def step(row, ib, isem, nib, nisem):
            pltpu.make_async_copy(idx_hbm.at[row], ib, isem).wait()
            pltpu.async_copy(idx_hbm.at[row + 1], nib, nisem)
            pltpu.async_copy(u_hbm.at[ib.at[0]], rows, sem).wait()
            pltpu.sync_copy(rows, acc.at[ib.at[1]], add=True)

        def body(k, _):
            row = tile_base + 2 * k
            step(row, ib0, isem0, ib1, isem1)
            step(row + 1, ib1, isem1, ib0, isem0)
            return 0
        lax.fori_loop(0, nblk // 2, body, 0)
        pltpu.make_async_copy(idx_hbm.at[tile_base], ib0, isem0).wait()
        plsc.subcore_barrier()
        pltpu.sync_copy(acc.at[pl.ds(sid * RPT, RPT)],
                        out_hbm.at[pl.ds(cid * N_PAD + sid * RPT, RPT)])

    return prop_kernel


_R = 1000


def _dinv(dp0, dp1):
    deg = 1.0 + dp0[:, 0:1] + dp1[:, 0:1]
    return lax.rsqrt(deg)


def _tc_first(x_ref, w_ref, dp0_ref, dp1_ref, u_ref):
    di = _dinv(dp0_ref[...], dp1_ref[...])
    u_ref[...] = di * jnp.dot(x_ref[...], w_ref[...],
                              preferred_element_type=jnp.float32)


def _tc_mid(s0_ref, s1_ref, u_ref, w_ref, b_ref, g_ref, be_ref,
            dp0_ref, dp1_ref, o_ref):
    di = _dinv(dp0_ref[...], dp1_ref[...])
    s = s0_ref[...] + s1_ref[...] + u_ref[...]
    out = di * s + b_ref[...]
    z = g_ref[...] * (out * (1.0 / jnp.sqrt(1.0 + EPS))) + be_ref[...]
    r = jnp.maximum(z, 0.0)
    o_ref[...] = di * jnp.dot(r, w_ref[...],
                              preferred_element_type=jnp.float32)


def _tc_last(s0_ref, s1_ref, u_ref, b_ref, dp0_ref, dp1_ref, o_ref):
    di = _dinv(dp0_ref[...], dp1_ref[...])
    o_ref[...] = di * (s0_ref[...] + s1_ref[...] + u_ref[...]) + b_ref[...]


def _row_spec(d):
    return pl.BlockSpec((_R, d), lambda i: (i, 0))


def _full_spec(r, c):
    return pl.BlockSpec((r, c), lambda i: (0, 0))


def kernel(x, edge_index, W1, b1, g1, be1, W2, b2, g2, be2, W3, b3):
    src = edge_index[0]
    dst = edge_index[1]
    pad = E_PAD - E
    src_p = jnp.concatenate(
        [src, jnp.zeros((pad,), jnp.int32)]).reshape(E_PAD // B, B)
    dst_pad = N + (jnp.arange(pad, dtype=jnp.int32) % (N_PAD - N))
    dst_p = jnp.concatenate([dst, dst_pad]).reshape(E_PAD // B, B)
    idx_p = jnp.concatenate(
        [jnp.stack([src_p, dst_p], axis=1),
         jnp.zeros((1, 2, B), jnp.int32)])

    deg_fn = _make_deg()
    prop128 = _make_prop(D_HID)
    prop64 = _make_prop(D_OUT)

    deg_out = deg_fn(dst_p)
    dp0 = deg_out[:N]
    dp1 = deg_out[N_PAD:N_PAD + N]

    grid = (N // _R,)
    dspec = pl.BlockSpec((_R, 16), lambda i: (i, 0))

    u1 = pl.pallas_call(
        _tc_first,
        grid=grid,
        in_specs=[_row_spec(D_HID), _full_spec(D_HID, D_HID), dspec, dspec],
        out_specs=_row_spec(D_HID),
        out_shape=jax.ShapeDtypeStruct((N, D_HID), jnp.float32),
    )(x, W1, dp0, dp1)

    p1 = prop128(idx_p, u1)
    u2 = pl.pallas_call(
        _tc_mid,
        grid=grid,
        in_specs=[_row_spec(D_HID), _row_spec(D_HID), _row_spec(D_HID),
                  _full_spec(D_HID, D_HID), _full_spec(1, D_HID),
                  _full_spec(1, D_HID), _full_spec(1, D_HID), dspec, dspec],
        out_specs=_row_spec(D_HID),
        out_shape=jax.ShapeDtypeStruct((N, D_HID), jnp.float32),
    )(p1[:N], p1[N_PAD:N_PAD + N], u1, W2, b1.reshape(1, -1),
      g1.reshape(1, -1), be1.reshape(1, -1), dp0, dp1)

    p2 = prop128(idx_p, u2)
    u3 = pl.pallas_call(
        _tc_mid,
        grid=grid,
        in_specs=[_row_spec(D_HID), _row_spec(D_HID), _row_spec(D_HID),
                  _full_spec(D_HID, D_OUT), _full_spec(1, D_HID),
                  _full_spec(1, D_HID), _full_spec(1, D_HID), dspec, dspec],
        out_specs=_row_spec(D_OUT),
        out_shape=jax.ShapeDtypeStruct((N, D_OUT), jnp.float32),
    )(p2[:N], p2[N_PAD:N_PAD + N], u2, W3, b2.reshape(1, -1),
      g2.reshape(1, -1), be2.reshape(1, -1), dp0, dp1)

    p3 = prop64(idx_p, u3)
    out = pl.pallas_call(
        _tc_last,
        grid=grid,
        in_specs=[_row_spec(D_OUT), _row_spec(D_OUT), _row_spec(D_OUT),
                  _full_spec(1, D_OUT), dspec, dspec],
        out_specs=_row_spec(D_OUT),
        out_shape=jax.ShapeDtypeStruct((N, D_OUT), jnp.float32),
    )(p3[:N], p3[N_PAD:N_PAD + N], u3, b3.reshape(1, -1), dp0, dp1)

    return out
